# baseline (speedup 1.0000x reference)
import functools

import jax
import jax.numpy as jnp
import numpy as np
from jax import lax
from jax.experimental import pallas as pl
from jax.experimental.pallas import tpu as pltpu
from jax.experimental.pallas import tpu_sc as plsc

S_LEVELS = 24
T_LEVELS = 19
S_TABLE = 2 ** 22
T_TABLE = 2 ** 18
F = 2
SPATIAL_RES = [int(16 * (1.38 ** l)) for l in range(S_LEVELS)]
TEMPORAL_RES = [int(8 * (1.5 ** l)) for l in range(T_LEVELS)]

B = 16384
NC = 2
NS = 16
NW = NC * NS
PTS = B // NW
NG = PTS // 16
S_DIM = 32
NLV = S_LEVELS + T_LEVELS
E_ALL = NLV * F + S_DIM


def _v(val, dtype):
    return jnp.full((16,), val, dtype)


def _enc_body(ct_hbm, sidx_hbm, semb_hbm, stab_hbm, ttab_hbm, res_hbm, ci_hbm,
              feat_hbm,
              cxb, cyb, czb, ctb, sidxb, selb, sbufT, resb, cib,
              idxb, wb, dstb, fbuf):
    wid = lax.axis_index("s") * NC + lax.axis_index("c")
    base = wid * PTS

    pltpu.sync_copy(ct_hbm.at[0, pl.ds(base, PTS)], cxb)
    pltpu.sync_copy(ct_hbm.at[1, pl.ds(base, PTS)], cyb)
    pltpu.sync_copy(ct_hbm.at[2, pl.ds(base, PTS)], czb)
    pltpu.sync_copy(ct_hbm.at[3, pl.ds(base, PTS)], ctb)
    pltpu.sync_copy(sidx_hbm.at[pl.ds(base, PTS)], sidxb)
    pltpu.sync_copy(res_hbm, resb)
    pltpu.sync_copy(ci_hbm, cib)

    iota = lax.broadcasted_iota(jnp.int32, (16,), 0)
    half = iota // 2
    parity = iota % 2
    iota2 = iota * 2
    iotaE = iota * E_ALL
    fidx0 = half * E_ALL + parity

    mask_s = _v(S_TABLE - 1, jnp.uint32)
    mask_t = _v(T_TABLE - 1, jnp.uint32)
    one_u = _v(1, jnp.uint32)
    one_f = _v(1.0, jnp.float32)
    p1 = _v(2654435761, jnp.uint32)
    p2 = _v(805459861, jnp.uint32)

    @pl.loop(0, NG)
    def _sidx(g):
        o = g * 16
        s32 = sidxb[pl.ds(o, 16)] * S_DIM
        for j in range(S_DIM):
            selb[pl.ds(j * PTS + o, 16)] = s32 + j

    pltpu.sync_copy(semb_hbm.at[selb], sbufT)

    @pl.loop(0, NG)
    def _strans(g):
        o = g * 16
        tgt = fbuf.at[pl.ds(o * E_ALL, 16 * E_ALL)]
        for j in range(S_DIM):
            v = sbufT[pl.ds(j * PTS + o, 16)]
            plsc.store_scatter(tgt, [iotaE + (NLV * F + j)], v)

    @pl.loop(0, S_LEVELS)
    def _spatial(l):
        res_v = resb[pl.ds(l * 16, 16)]
        base2_v = cib[pl.ds(l * 16, 16)]
        colv = cib[pl.ds((NLV + l) * 16, 16)]

        @pl.loop(0, NG)
        def _idx(g):
            o = g * 16
            x = cxb[pl.ds(o, 16)] * res_v
            y = cyb[pl.ds(o, 16)] * res_v
            z = czb[pl.ds(o, 16)] * res_v
            xi = x.astype(jnp.uint32)
            yi = y.astype(jnp.uint32)
            zi = z.astype(jnp.uint32)
            wx = x - xi.astype(jnp.float32)
            wy = y - yi.astype(jnp.float32)
            wz = z - zi.astype(jnp.float32)
            axs = (xi, xi + one_u)
            by0 = yi * p1
            bys = (by0, by0 + p1)
            cz0 = zi * p2
            czs = (cz0, cz0 + p2)
            ux = one_f - wx
            uy = one_f - wy
            uz = one_f - wz
            wxy = (ux * uy, wx * uy, ux * wy, wx * wy)
            for c in range(8):
                bx, by_, bz = c & 1, (c >> 1) & 1, (c >> 2) & 1
                h = axs[bx] ^ bys[by_] ^ czs[bz]
                e0 = (h & mask_s).astype(jnp.int32) * 2 + base2_v
                tgt = idxb.at[pl.ds((c * PTS + o) * 2, 32)]
                plsc.store_scatter(tgt, [iota2], e0)
                plsc.store_scatter(tgt, [iota2 + 1], e0 + 1)
                wb[pl.ds(c * PTS + o, 16)] = wxy[bx + 2 * by_] * (wz if bz else uz)

        pltpu.sync_copy(stab_hbm.at[idxb], dstb)

        @pl.loop(0, 2 * NG)
        def _acc(g):
            p8 = g * 8
            acc = jnp.zeros((16,), jnp.float32)
            for c in range(8):
                v = dstb[pl.ds((c * PTS + p8) * 2, 16)]
                w = plsc.load_gather(wb.at[pl.ds(c * PTS + p8, 8)], [half])
                acc = acc + v * w
            plsc.store_scatter(fbuf.at[pl.ds(p8 * E_ALL, 8 * E_ALL)],
                               [fidx0 + colv], acc)

    @pl.loop(0, T_LEVELS)
    def _temporal(l):
        res_v = resb[pl.ds((S_LEVELS + l) * 16, 16)]
        base2_v = cib[pl.ds((S_LEVELS + l) * 16, 16)]
        colv = cib[pl.ds((NLV + S_LEVELS + l) * 16, 16)]

        @pl.loop(0, NG)
        def _idx(g):
            o = g * 16
            x = ctb[pl.ds(o, 16)] * res_v
            i0 = x.astype(jnp.uint32)
            w = x - i0.astype(jnp.float32)
            h0 = i0 * p1
            h1 = h0 + p1
            e0 = (h0 & mask_t).astype(jnp.int32) * 2 + base2_v
            e1 = (h1 & mask_t).astype(jnp.int32) * 2 + base2_v
            tgt = idxb.at[pl.ds(2 * o, 32)]
            plsc.store_scatter(tgt, [iota2], e0)
            plsc.store_scatter(tgt, [iota2 + 1], e0 + 1)
            tgt2 = idxb.at[pl.ds(2 * (PTS + o), 32)]
            plsc.store_scatter(tgt2, [iota2], e1)
            plsc.store_scatter(tgt2, [iota2 + 1], e1 + 1)
            wb[pl.ds(o, 16)] = w

        pltpu.sync_copy(ttab_hbm.at[idxb.at[pl.ds(0, 4 * PTS)]], dstb.at[pl.ds(0, 4 * PTS)])

        @pl.loop(0, 2 * NG)
        def _acc(g):
            p8 = g * 8
            w = plsc.load_gather(wb.at[pl.ds(p8, 8)], [half])
            v0 = dstb[pl.ds(p8 * 2, 16)]
            v1 = dstb[pl.ds((PTS + p8) * 2, 16)]
            plsc.store_scatter(fbuf.at[pl.ds(p8 * E_ALL, 8 * E_ALL)],
                               [fidx0 + colv], v0 * (one_f - w) + v1 * w)

    pltpu.sync_copy(fbuf, feat_hbm.at[pl.ds(base * E_ALL, PTS * E_ALL)])


_encode = functools.partial(
    pl.kernel,
    out_type=jax.ShapeDtypeStruct((B * E_ALL,), jnp.float32),
    mesh=plsc.VectorSubcoreMesh(core_axis_name="c", subcore_axis_name="s"),
    compiler_params=pltpu.CompilerParams(needs_layout_passes=False),
    scratch_types=[
        pltpu.VMEM((PTS,), jnp.float32),
        pltpu.VMEM((PTS,), jnp.float32),
        pltpu.VMEM((PTS,), jnp.float32),
        pltpu.VMEM((PTS,), jnp.float32),
        pltpu.VMEM((PTS,), jnp.int32),
        pltpu.VMEM((S_DIM * PTS,), jnp.int32),
        pltpu.VMEM((S_DIM * PTS,), jnp.float32),
        pltpu.VMEM((NLV * 16,), jnp.float32),
        pltpu.VMEM((2 * NLV * 16,), jnp.int32),
        pltpu.VMEM((8 * PTS * F,), jnp.int32),
        pltpu.VMEM((8 * PTS,), jnp.float32),
        pltpu.VMEM((8 * PTS * F,), jnp.float32),
        pltpu.VMEM((PTS * E_ALL,), jnp.float32),
    ],
)(_enc_body)


def _mlp_body(x_ref, w1_ref, b1_ref, w2_ref, b2_ref, w3_ref, b3_ref, o_ref):
    h = jnp.dot(x_ref[...], w1_ref[...], preferred_element_type=jnp.float32)
    h = jnp.maximum(h + b1_ref[...], 0.0)
    h = jnp.maximum(
        jnp.dot(h, w2_ref[...], preferred_element_type=jnp.float32) + b2_ref[...], 0.0)
    o_ref[...] = jnp.dot(h, w3_ref[...], preferred_element_type=jnp.float32) + b3_ref[...]


_BM = 2048


def _mlp(x, w1, b1, w2, b2, w3, b3):
    grid = (B // _BM,)
    return pl.pallas_call(
        _mlp_body,
        grid=grid,
        in_specs=[
            pl.BlockSpec((_BM, E_ALL), lambda i: (i, 0)),
            pl.BlockSpec((E_ALL, 256), lambda i: (0, 0)),
            pl.BlockSpec((1, 256), lambda i: (0, 0)),
            pl.BlockSpec((256, 128), lambda i: (0, 0)),
            pl.BlockSpec((1, 128), lambda i: (0, 0)),
            pl.BlockSpec((128, 1), lambda i: (0, 0)),
            pl.BlockSpec((1, 1), lambda i: (0, 0)),
        ],
        out_specs=pl.BlockSpec((_BM, 1), lambda i: (i, 0)),
        out_shape=jax.ShapeDtypeStruct((B, 1), jnp.float32),
    )(x, w1, b1, w2, b2, w3, b3)


_RES = np.repeat(np.array(SPATIAL_RES + TEMPORAL_RES, np.float32), 16)
_CI = np.repeat(np.array(
    [2 * l * S_TABLE for l in range(S_LEVELS)]
    + [2 * l * T_TABLE for l in range(T_LEVELS)]
    + [2 * l for l in range(S_LEVELS)]
    + [2 * S_LEVELS + 2 * l for l in range(T_LEVELS)], np.int32), 16)


def kernel(coords, species_idx, species_emb, spatial_tables, temporal_tables,
           W1, b1, W2, b2, W3, b3):
    coords_t = coords.T
    stab = spatial_tables.reshape(S_LEVELS * S_TABLE * F)
    ttab = temporal_tables.reshape(T_LEVELS * T_TABLE * F)
    semb = species_emb.reshape(species_emb.shape[0] * S_DIM)
    res = jnp.asarray(_RES)
    ci = jnp.asarray(_CI)
    feat = _encode(coords_t, species_idx, semb, stab, ttab, res, ci)
    x = feat.reshape(B, E_ALL)
    y = _mlp(x, W1, b1.reshape(1, 256), W2, b2.reshape(1, 128),
             W3, b3.reshape(1, 1))
    return y[:, 0]


# speedup vs baseline: 105.0832x; 105.0832x over previous
import functools

import jax
import jax.numpy as jnp
import numpy as np
from jax import lax
from jax.experimental import pallas as pl
from jax.experimental.pallas import tpu as pltpu
from jax.experimental.pallas import tpu_sc as plsc

S_LEVELS = 24
T_LEVELS = 19
S_TABLE = 2 ** 22
T_TABLE = 2 ** 18
F = 2
SPATIAL_RES = [int(16 * (1.38 ** l)) for l in range(S_LEVELS)]
TEMPORAL_RES = [int(8 * (1.5 ** l)) for l in range(T_LEVELS)]

B = 16384
NC = 2
NS = 16
NW = NC * NS
PTS = B // NW
NG = PTS // 16
S_DIM = 32
NLV = S_LEVELS + T_LEVELS
E_ALL = NLV * F + S_DIM


def _v(val, dtype):
    return jnp.full((16,), val, dtype)


def _enc_body(ct_hbm, sidx_hbm, semb_hbm, stab_hbm, ttab_hbm, res_hbm, ci_hbm,
              feat_hbm,
              cxb, cyb, czb, ctb, sidxb, selb, sbufT, resb, cib,
              idxb, wb, dstb, fbuf):
    wid = lax.axis_index("s") * NC + lax.axis_index("c")
    base = wid * PTS

    pltpu.sync_copy(ct_hbm.at[0, pl.ds(base, PTS)], cxb)
    pltpu.sync_copy(ct_hbm.at[1, pl.ds(base, PTS)], cyb)
    pltpu.sync_copy(ct_hbm.at[2, pl.ds(base, PTS)], czb)
    pltpu.sync_copy(ct_hbm.at[3, pl.ds(base, PTS)], ctb)
    pltpu.sync_copy(sidx_hbm.at[pl.ds(base, PTS)], sidxb)
    pltpu.sync_copy(res_hbm, resb)
    pltpu.sync_copy(ci_hbm, cib)

    iota = lax.broadcasted_iota(jnp.int32, (16,), 0)
    half = iota // 2
    parity = iota % 2
    iota2 = iota * 2
    iotaE = iota * E_ALL
    fidx0 = half * E_ALL + parity

    mask_s = _v(S_TABLE - 1, jnp.uint32)
    mask_t = _v(T_TABLE - 1, jnp.uint32)
    one_u = _v(1, jnp.uint32)
    one_f = _v(1.0, jnp.float32)
    p1 = _v(2654435761, jnp.uint32)
    p2 = _v(805459861, jnp.uint32)

    @pl.loop(0, NG)
    def _sidx(g):
        o = g * 16
        s32 = sidxb[pl.ds(o, 16)] * S_DIM
        for j in range(S_DIM):
            selb[pl.ds(j * PTS + o, 16)] = s32 + j

    pltpu.sync_copy(semb_hbm.at[selb], sbufT)

    @pl.loop(0, NG)
    def _strans(g):
        o = g * 16
        tgt = fbuf.at[pl.ds(o * E_ALL, 16 * E_ALL)]
        for j in range(S_DIM):
            v = sbufT[pl.ds(j * PTS + o, 16)]
            plsc.store_scatter(tgt, [iotaE + (NLV * F + j)], v)

    @pl.loop(0, S_LEVELS)
    def _spatial(l):
        res_v = resb[pl.ds(l * 16, 16)]
        base2_v = cib[pl.ds(l * 16, 16)]
        colv = cib[pl.ds((NLV + l) * 16, 16)]

        @pl.loop(0, NG)
        def _idx(g):
            o = g * 16
            x = cxb[pl.ds(o, 16)] * res_v
            y = cyb[pl.ds(o, 16)] * res_v
            z = czb[pl.ds(o, 16)] * res_v
            xi = x.astype(jnp.uint32)
            yi = y.astype(jnp.uint32)
            zi = z.astype(jnp.uint32)
            wx = x - xi.astype(jnp.float32)
            wy = y - yi.astype(jnp.float32)
            wz = z - zi.astype(jnp.float32)
            axs = (xi, xi + one_u)
            by0 = yi * p1
            bys = (by0, by0 + p1)
            cz0 = zi * p2
            czs = (cz0, cz0 + p2)
            ux = one_f - wx
            uy = one_f - wy
            uz = one_f - wz
            wxy = (ux * uy, wx * uy, ux * wy, wx * wy)
            for c in range(8):
                bx, by_, bz = c & 1, (c >> 1) & 1, (c >> 2) & 1
                h = axs[bx] ^ bys[by_] ^ czs[bz]
                r = h & mask_s
                e0 = (((r >> 7) << 8) + (r & 127)).astype(jnp.int32) + base2_v
                tgt = idxb.at[pl.ds((c * PTS + o) * 2, 32)]
                plsc.store_scatter(tgt, [iota2], e0)
                plsc.store_scatter(tgt, [iota2 + 1], e0 + 128)
                wb[pl.ds(c * PTS + o, 16)] = wxy[bx + 2 * by_] * (wz if bz else uz)

        pltpu.sync_copy(stab_hbm.at[idxb], dstb)

        @pl.loop(0, 2 * NG)
        def _acc(g):
            p8 = g * 8
            acc = jnp.zeros((16,), jnp.float32)
            for c in range(8):
                v = dstb[pl.ds((c * PTS + p8) * 2, 16)]
                w = plsc.load_gather(wb.at[pl.ds(c * PTS + p8, 8)], [half])
                acc = acc + v * w
            plsc.store_scatter(fbuf.at[pl.ds(p8 * E_ALL, 8 * E_ALL)],
                               [fidx0 + colv], acc)

    @pl.loop(0, T_LEVELS)
    def _temporal(l):
        res_v = resb[pl.ds((S_LEVELS + l) * 16, 16)]
        base2_v = cib[pl.ds((S_LEVELS + l) * 16, 16)]
        colv = cib[pl.ds((NLV + S_LEVELS + l) * 16, 16)]

        @pl.loop(0, NG)
        def _idx(g):
            o = g * 16
            x = ctb[pl.ds(o, 16)] * res_v
            i0 = x.astype(jnp.uint32)
            w = x - i0.astype(jnp.float32)
            h0 = i0 * p1
            h1 = h0 + p1
            r0 = h0 & mask_t
            r1 = h1 & mask_t
            e0 = (((r0 >> 7) << 8) + (r0 & 127)).astype(jnp.int32) + base2_v
            e1 = (((r1 >> 7) << 8) + (r1 & 127)).astype(jnp.int32) + base2_v
            tgt = idxb.at[pl.ds(2 * o, 32)]
            plsc.store_scatter(tgt, [iota2], e0)
            plsc.store_scatter(tgt, [iota2 + 1], e0 + 128)
            tgt2 = idxb.at[pl.ds(2 * (PTS + o), 32)]
            plsc.store_scatter(tgt2, [iota2], e1)
            plsc.store_scatter(tgt2, [iota2 + 1], e1 + 128)
            wb[pl.ds(o, 16)] = w

        pltpu.sync_copy(ttab_hbm.at[idxb.at[pl.ds(0, 4 * PTS)]], dstb.at[pl.ds(0, 4 * PTS)])

        @pl.loop(0, 2 * NG)
        def _acc(g):
            p8 = g * 8
            w = plsc.load_gather(wb.at[pl.ds(p8, 8)], [half])
            v0 = dstb[pl.ds(p8 * 2, 16)]
            v1 = dstb[pl.ds((PTS + p8) * 2, 16)]
            plsc.store_scatter(fbuf.at[pl.ds(p8 * E_ALL, 8 * E_ALL)],
                               [fidx0 + colv], v0 * (one_f - w) + v1 * w)

    pltpu.sync_copy(fbuf, feat_hbm.at[pl.ds(base * E_ALL, PTS * E_ALL)])


_encode = functools.partial(
    pl.kernel,
    out_type=jax.ShapeDtypeStruct((B * E_ALL,), jnp.float32),
    mesh=plsc.VectorSubcoreMesh(core_axis_name="c", subcore_axis_name="s"),
    compiler_params=pltpu.CompilerParams(needs_layout_passes=False),
    scratch_types=[
        pltpu.VMEM((PTS,), jnp.float32),
        pltpu.VMEM((PTS,), jnp.float32),
        pltpu.VMEM((PTS,), jnp.float32),
        pltpu.VMEM((PTS,), jnp.float32),
        pltpu.VMEM((PTS,), jnp.int32),
        pltpu.VMEM((S_DIM * PTS,), jnp.int32),
        pltpu.VMEM((S_DIM * PTS,), jnp.float32),
        pltpu.VMEM((NLV * 16,), jnp.float32),
        pltpu.VMEM((2 * NLV * 16,), jnp.int32),
        pltpu.VMEM((8 * PTS * F,), jnp.int32),
        pltpu.VMEM((8 * PTS,), jnp.float32),
        pltpu.VMEM((8 * PTS * F,), jnp.float32),
        pltpu.VMEM((PTS * E_ALL,), jnp.float32),
    ],
)(_enc_body)


def _mlp_body(x_ref, w1_ref, b1_ref, w2_ref, b2_ref, w3_ref, b3_ref, o_ref):
    h = jnp.dot(x_ref[...], w1_ref[...], preferred_element_type=jnp.float32)
    h = jnp.maximum(h + b1_ref[...], 0.0)
    h = jnp.maximum(
        jnp.dot(h, w2_ref[...], preferred_element_type=jnp.float32) + b2_ref[...], 0.0)
    o_ref[...] = jnp.dot(h, w3_ref[...], preferred_element_type=jnp.float32) + b3_ref[...]


_BM = 2048


def _mlp(x, w1, b1, w2, b2, w3, b3):
    grid = (B // _BM,)
    return pl.pallas_call(
        _mlp_body,
        grid=grid,
        in_specs=[
            pl.BlockSpec((_BM, E_ALL), lambda i: (i, 0)),
            pl.BlockSpec((E_ALL, 256), lambda i: (0, 0)),
            pl.BlockSpec((1, 256), lambda i: (0, 0)),
            pl.BlockSpec((256, 128), lambda i: (0, 0)),
            pl.BlockSpec((1, 128), lambda i: (0, 0)),
            pl.BlockSpec((128, 1), lambda i: (0, 0)),
            pl.BlockSpec((1, 1), lambda i: (0, 0)),
        ],
        out_specs=pl.BlockSpec((_BM, 1), lambda i: (i, 0)),
        out_shape=jax.ShapeDtypeStruct((B, 1), jnp.float32),
    )(x, w1, b1, w2, b2, w3, b3)


_RES = np.repeat(np.array(SPATIAL_RES + TEMPORAL_RES, np.float32), 16)
_CI = np.repeat(np.array(
    [2 * l * S_TABLE for l in range(S_LEVELS)]
    + [2 * l * T_TABLE for l in range(T_LEVELS)]
    + [2 * l for l in range(S_LEVELS)]
    + [2 * S_LEVELS + 2 * l for l in range(T_LEVELS)], np.int32), 16)


def kernel(coords, species_idx, species_emb, spatial_tables, temporal_tables,
           W1, b1, W2, b2, W3, b3):
    coords_t = coords.T
    stab = (spatial_tables.reshape(S_LEVELS, S_TABLE // 128, 128, F)
            .transpose(0, 1, 3, 2).reshape(S_LEVELS * S_TABLE * F))
    ttab = (temporal_tables.reshape(T_LEVELS, T_TABLE // 128, 128, F)
            .transpose(0, 1, 3, 2).reshape(T_LEVELS * T_TABLE * F))
    semb = species_emb.reshape(species_emb.shape[0] * S_DIM)
    res = jnp.asarray(_RES)
    ci = jnp.asarray(_CI)
    feat = _encode(coords_t, species_idx, semb, stab, ttab, res, ci)
    x = feat.reshape(B, E_ALL)
    y = _mlp(x, W1, b1.reshape(1, 256), W2, b2.reshape(1, 128),
             W3, b3.reshape(1, 1))
    return y[:, 0]
